# initial kernel scaffold (unmeasured)
import jax
import jax.numpy as jnp
from jax import lax
from jax.experimental import pallas as pl
from jax.experimental.pallas import tpu as pltpu

N_DEV = 4


def kernel(A, B):
    A = A.astype(jnp.bfloat16)
    B = B.astype(jnp.bfloat16)
    m_per, k = A.shape
    n = B.shape[1]

    def body(a_ref, b_ref, out_ref, comm_ref, send_sems, recv_sems):
        my = lax.axis_index("i")
        left = lax.rem(my + N_DEV - 1, N_DEV)
        right = lax.rem(my + 1, N_DEV)

        barrier_sem = pltpu.get_barrier_semaphore()
        pl.semaphore_signal(
            barrier_sem, inc=1, device_id=(left,),
            device_id_type=pl.DeviceIdType.MESH,
        )
        pl.semaphore_signal(
            barrier_sem, inc=1, device_id=(right,),
            device_id_type=pl.DeviceIdType.MESH,
        )
        pl.semaphore_wait(barrier_sem, 2)

        rdmas = []
        for h in range(1, N_DEV):
            src = a_ref if h == 1 else comm_ref.at[h - 2]
            rdmas.append(
                pltpu.make_async_remote_copy(
                    src_ref=src,
                    dst_ref=comm_ref.at[h - 1],
                    send_sem=send_sems.at[h - 1],
                    recv_sem=recv_sems.at[h - 1],
                    device_id=(right,),
                    device_id_type=pl.DeviceIdType.MESH,
                )
            )

        rdmas[0].start()

        acc = jnp.dot(a_ref[...], b_ref[...], preferred_element_type=jnp.float32)
        out_ref[pl.ds(my * m_per, m_per), :] = acc.astype(out_ref.dtype)

        for h in range(1, N_DEV):
            rdmas[h - 1].wait_recv()
            if h < N_DEV - 1:
                rdmas[h].start()
            origin = lax.rem(my + N_DEV - h, N_DEV)
            acc = jnp.dot(
                comm_ref[h - 1], b_ref[...], preferred_element_type=jnp.float32
            )
            out_ref[pl.ds(origin * m_per, m_per), :] = acc.astype(out_ref.dtype)

        for h in range(1, N_DEV):
            rdmas[h - 1].wait_send()

    return pl.pallas_call(
        body,
        out_shape=jax.ShapeDtypeStruct((N_DEV * m_per, n), jnp.bfloat16),
        in_specs=[
            pl.BlockSpec(memory_space=pltpu.VMEM),
            pl.BlockSpec(memory_space=pltpu.VMEM),
        ],
        out_specs=pl.BlockSpec(memory_space=pltpu.VMEM),
        scratch_shapes=[
            pltpu.VMEM((N_DEV - 1, m_per, k), jnp.bfloat16),
            pltpu.SemaphoreType.DMA((N_DEV - 1,)),
            pltpu.SemaphoreType.DMA((N_DEV - 1,)),
        ],
        compiler_params=pltpu.CompilerParams(collective_id=0),
    )(A, B)


# baseline (device time: 197968 ns/iter reference)
import jax
import jax.numpy as jnp
from jax import lax
from jax.experimental import pallas as pl
from jax.experimental.pallas import tpu as pltpu

N_DEV = 4


def kernel(A, B):
    A = A.astype(jnp.bfloat16)
    B = B.astype(jnp.bfloat16)
    m_per, k = A.shape
    n = B.shape[1]

    def body(a_ref, b_ref, out_ref, comm_ref, send_sems, recv_sems):
        my = lax.axis_index("i")
        left = lax.rem(my + N_DEV - 1, N_DEV)
        right = lax.rem(my + 1, N_DEV)

        barrier_sem = pltpu.get_barrier_semaphore()
        pl.semaphore_signal(
            barrier_sem, inc=1, device_id=(left,),
            device_id_type=pl.DeviceIdType.MESH,
        )
        pl.semaphore_signal(
            barrier_sem, inc=1, device_id=(right,),
            device_id_type=pl.DeviceIdType.MESH,
        )
        pl.semaphore_wait(barrier_sem, 2)

        rdmas = []
        for h in range(1, N_DEV):
            src = a_ref if h == 1 else comm_ref.at[h - 2]
            rdmas.append(
                pltpu.make_async_remote_copy(
                    src_ref=src,
                    dst_ref=comm_ref.at[h - 1],
                    send_sem=send_sems.at[h - 1],
                    recv_sem=recv_sems.at[h - 1],
                    device_id=(right,),
                    device_id_type=pl.DeviceIdType.MESH,
                )
            )

        rdmas[0].start()

        acc = jnp.dot(a_ref[...], b_ref[...], preferred_element_type=jnp.float32)
        out_ref[pl.ds(my * m_per, m_per), :] = acc.astype(out_ref.dtype)

        for h in range(1, N_DEV):
            rdmas[h - 1].wait_recv()
            if h < N_DEV - 1:
                rdmas[h].start()
            origin = lax.rem(my + N_DEV - h, N_DEV)
            acc = jnp.dot(
                comm_ref[h - 1], b_ref[...], preferred_element_type=jnp.float32
            )
            out_ref[pl.ds(origin * m_per, m_per), :] = acc.astype(out_ref.dtype)

        for h in range(1, N_DEV):
            rdmas[h - 1].wait_send()

    return pl.pallas_call(
        body,
        out_shape=jax.ShapeDtypeStruct((N_DEV * m_per, n), jnp.bfloat16),
        in_specs=[
            pl.BlockSpec(memory_space=pltpu.VMEM),
            pl.BlockSpec(memory_space=pltpu.VMEM),
        ],
        out_specs=pl.BlockSpec(memory_space=pltpu.VMEM),
        scratch_shapes=[
            pltpu.VMEM((N_DEV - 1, m_per, k), jnp.bfloat16),
            pltpu.SemaphoreType.DMA((N_DEV - 1,)),
            pltpu.SemaphoreType.DMA((N_DEV - 1,)),
        ],
        compiler_params=pltpu.CompilerParams(
            collective_id=0, vmem_limit_bytes=110 * 1024 * 1024
        ),
    )(A, B)


# device time: 138324 ns/iter; 1.4312x vs baseline; 1.4312x over previous
import jax
import jax.numpy as jnp
from jax import lax
from jax.experimental import pallas as pl
from jax.experimental.pallas import tpu as pltpu

N_DEV = 4


def kernel(A, B):
    A = A.astype(jnp.bfloat16)
    B = B.astype(jnp.bfloat16)
    m_per, k = A.shape
    n = B.shape[1]
    half = m_per // 2

    def body(a_ref, b_ref, out_ref, cl_ref, cr_ref, cf_ref, send_sems, recv_sems):
        my = lax.axis_index("i")
        left = lax.rem(my + N_DEV - 1, N_DEV)
        right = lax.rem(my + 1, N_DEV)

        barrier_sem = pltpu.get_barrier_semaphore()
        pl.semaphore_signal(
            barrier_sem, inc=1, device_id=(left,),
            device_id_type=pl.DeviceIdType.MESH,
        )
        pl.semaphore_signal(
            barrier_sem, inc=1, device_id=(right,),
            device_id_type=pl.DeviceIdType.MESH,
        )
        pl.semaphore_wait(barrier_sem, 2)

        r1 = pltpu.make_async_remote_copy(
            src_ref=a_ref, dst_ref=cl_ref,
            send_sem=send_sems.at[0], recv_sem=recv_sems.at[0],
            device_id=(right,), device_id_type=pl.DeviceIdType.MESH,
        )
        l1 = pltpu.make_async_remote_copy(
            src_ref=a_ref, dst_ref=cr_ref,
            send_sem=send_sems.at[1], recv_sem=recv_sems.at[1],
            device_id=(left,), device_id_type=pl.DeviceIdType.MESH,
        )
        r2 = pltpu.make_async_remote_copy(
            src_ref=cl_ref.at[pl.ds(0, half), :],
            dst_ref=cf_ref.at[pl.ds(0, half), :],
            send_sem=send_sems.at[2], recv_sem=recv_sems.at[2],
            device_id=(right,), device_id_type=pl.DeviceIdType.MESH,
        )
        l2 = pltpu.make_async_remote_copy(
            src_ref=cr_ref.at[pl.ds(half, half), :],
            dst_ref=cf_ref.at[pl.ds(half, half), :],
            send_sem=send_sems.at[3], recv_sem=recv_sems.at[3],
            device_id=(left,), device_id_type=pl.DeviceIdType.MESH,
        )

        r1.start()
        l1.start()

        b = b_ref[...]
        acc = jnp.dot(a_ref[...], b, preferred_element_type=jnp.float32)
        out_ref[pl.ds(my * m_per, m_per), :] = acc.astype(out_ref.dtype)

        r1.wait_recv()
        r2.start()
        acc = jnp.dot(cl_ref[...], b, preferred_element_type=jnp.float32)
        out_ref[pl.ds(left * m_per, m_per), :] = acc.astype(out_ref.dtype)

        l1.wait_recv()
        l2.start()
        acc = jnp.dot(cr_ref[...], b, preferred_element_type=jnp.float32)
        out_ref[pl.ds(right * m_per, m_per), :] = acc.astype(out_ref.dtype)

        far = lax.rem(my + 2, N_DEV)
        r2.wait_recv()
        l2.wait_recv()
        acc = jnp.dot(cf_ref[...], b, preferred_element_type=jnp.float32)
        out_ref[pl.ds(far * m_per, m_per), :] = acc.astype(out_ref.dtype)

        r1.wait_send()
        l1.wait_send()
        r2.wait_send()
        l2.wait_send()

    return pl.pallas_call(
        body,
        out_shape=jax.ShapeDtypeStruct((N_DEV * m_per, n), jnp.bfloat16),
        in_specs=[
            pl.BlockSpec(memory_space=pltpu.VMEM),
            pl.BlockSpec(memory_space=pltpu.VMEM),
        ],
        out_specs=pl.BlockSpec(memory_space=pltpu.VMEM),
        scratch_shapes=[
            pltpu.VMEM((m_per, k), jnp.bfloat16),
            pltpu.VMEM((m_per, k), jnp.bfloat16),
            pltpu.VMEM((m_per, k), jnp.bfloat16),
            pltpu.SemaphoreType.DMA((4,)),
            pltpu.SemaphoreType.DMA((4,)),
        ],
        compiler_params=pltpu.CompilerParams(
            collective_id=0, vmem_limit_bytes=110 * 1024 * 1024
        ),
    )(A, B)


# device time: 128244 ns/iter; 1.5437x vs baseline; 1.0786x over previous
import jax
import jax.numpy as jnp
from jax import lax
from jax.experimental import pallas as pl
from jax.experimental.pallas import tpu as pltpu

N_DEV = 4


def kernel(A, B):
    A = A.astype(jnp.bfloat16)
    B = B.astype(jnp.bfloat16)
    m_per, k = A.shape
    n = B.shape[1]
    half = m_per // 2

    def body(a_ref, b_ref, out_ref, cl_ref, cr_ref, cf_ref, send_sems, recv_sems):
        my = lax.axis_index("i")
        left = lax.rem(my + N_DEV - 1, N_DEV)
        right = lax.rem(my + 1, N_DEV)

        barrier_sem = pltpu.get_barrier_semaphore()
        pl.semaphore_signal(
            barrier_sem, inc=1, device_id=(left,),
            device_id_type=pl.DeviceIdType.MESH,
        )
        pl.semaphore_signal(
            barrier_sem, inc=1, device_id=(right,),
            device_id_type=pl.DeviceIdType.MESH,
        )
        pl.semaphore_wait(barrier_sem, 2)

        r1 = pltpu.make_async_remote_copy(
            src_ref=a_ref, dst_ref=cl_ref,
            send_sem=send_sems.at[0], recv_sem=recv_sems.at[0],
            device_id=(right,), device_id_type=pl.DeviceIdType.MESH,
        )
        l1 = pltpu.make_async_remote_copy(
            src_ref=a_ref, dst_ref=cr_ref,
            send_sem=send_sems.at[1], recv_sem=recv_sems.at[1],
            device_id=(left,), device_id_type=pl.DeviceIdType.MESH,
        )
        r2 = pltpu.make_async_remote_copy(
            src_ref=cl_ref.at[pl.ds(0, half), :],
            dst_ref=cf_ref.at[pl.ds(0, half), :],
            send_sem=send_sems.at[2], recv_sem=recv_sems.at[2],
            device_id=(right,), device_id_type=pl.DeviceIdType.MESH,
        )
        l2 = pltpu.make_async_remote_copy(
            src_ref=cr_ref.at[pl.ds(half, half), :],
            dst_ref=cf_ref.at[pl.ds(half, half), :],
            send_sem=send_sems.at[3], recv_sem=recv_sems.at[3],
            device_id=(left,), device_id_type=pl.DeviceIdType.MESH,
        )

        r1.start()
        l1.start()

        b = b_ref[...]
        acc = jnp.dot(a_ref[...], b, preferred_element_type=jnp.float32)
        out_ref[pl.ds(my * m_per, m_per), :] = acc.astype(out_ref.dtype)

        r1.wait_recv()
        r2.start()
        l1.wait_recv()
        l2.start()

        acc = jnp.dot(cl_ref[...], b, preferred_element_type=jnp.float32)
        out_ref[pl.ds(left * m_per, m_per), :] = acc.astype(out_ref.dtype)
        acc = jnp.dot(cr_ref[...], b, preferred_element_type=jnp.float32)
        out_ref[pl.ds(right * m_per, m_per), :] = acc.astype(out_ref.dtype)

        far = lax.rem(my + 2, N_DEV)
        r2.wait_recv()
        l2.wait_recv()
        acc = jnp.dot(cf_ref[...], b, preferred_element_type=jnp.float32)
        out_ref[pl.ds(far * m_per, m_per), :] = acc.astype(out_ref.dtype)

        r1.wait_send()
        l1.wait_send()
        r2.wait_send()
        l2.wait_send()

    return pl.pallas_call(
        body,
        out_shape=jax.ShapeDtypeStruct((N_DEV * m_per, n), jnp.bfloat16),
        in_specs=[
            pl.BlockSpec(memory_space=pltpu.VMEM),
            pl.BlockSpec(memory_space=pltpu.VMEM),
        ],
        out_specs=pl.BlockSpec(memory_space=pltpu.VMEM),
        scratch_shapes=[
            pltpu.VMEM((m_per, k), jnp.bfloat16),
            pltpu.VMEM((m_per, k), jnp.bfloat16),
            pltpu.VMEM((m_per, k), jnp.bfloat16),
            pltpu.SemaphoreType.DMA((4,)),
            pltpu.SemaphoreType.DMA((4,)),
        ],
        compiler_params=pltpu.CompilerParams(
            collective_id=0, vmem_limit_bytes=110 * 1024 * 1024
        ),
    )(A, B)


# device time: 118074 ns/iter; 1.6766x vs baseline; 1.0861x over previous
import jax
import jax.numpy as jnp
from jax import lax
from jax.experimental import pallas as pl
from jax.experimental.pallas import tpu as pltpu

N_DEV = 4


def kernel(A, B):
    B = B.astype(jnp.bfloat16)
    m_per, k = A.shape
    n = B.shape[1]
    half = m_per // 2

    def body(a32_ref, b_ref, out_ref, a_ref, cl_ref, cr_ref, cf_ref,
             stage_ref, send_sems, recv_sems, store_sems):
        my = lax.axis_index("i")
        left = lax.rem(my + N_DEV - 1, N_DEV)
        right = lax.rem(my + 1, N_DEV)

        a_ref[...] = a32_ref[...].astype(jnp.bfloat16)

        barrier_sem = pltpu.get_barrier_semaphore()
        pl.semaphore_signal(
            barrier_sem, inc=1, device_id=(left,),
            device_id_type=pl.DeviceIdType.MESH,
        )
        pl.semaphore_signal(
            barrier_sem, inc=1, device_id=(right,),
            device_id_type=pl.DeviceIdType.MESH,
        )
        pl.semaphore_wait(barrier_sem, 2)

        r1 = pltpu.make_async_remote_copy(
            src_ref=a_ref, dst_ref=cl_ref,
            send_sem=send_sems.at[0], recv_sem=recv_sems.at[0],
            device_id=(right,), device_id_type=pl.DeviceIdType.MESH,
        )
        l1 = pltpu.make_async_remote_copy(
            src_ref=a_ref, dst_ref=cr_ref,
            send_sem=send_sems.at[1], recv_sem=recv_sems.at[1],
            device_id=(left,), device_id_type=pl.DeviceIdType.MESH,
        )
        r2 = pltpu.make_async_remote_copy(
            src_ref=cl_ref.at[pl.ds(0, half), :],
            dst_ref=cf_ref.at[pl.ds(0, half), :],
            send_sem=send_sems.at[2], recv_sem=recv_sems.at[2],
            device_id=(right,), device_id_type=pl.DeviceIdType.MESH,
        )
        l2 = pltpu.make_async_remote_copy(
            src_ref=cr_ref.at[pl.ds(half, half), :],
            dst_ref=cf_ref.at[pl.ds(half, half), :],
            send_sem=send_sems.at[3], recv_sem=recv_sems.at[3],
            device_id=(left,), device_id_type=pl.DeviceIdType.MESH,
        )

        r1.start()
        l1.start()

        b = b_ref[...]

        def compute_store(src_ref, origin, slot):
            stage_ref[slot] = jnp.dot(
                src_ref[...], b, preferred_element_type=jnp.float32
            ).astype(jnp.bfloat16)
            st = pltpu.make_async_copy(
                stage_ref.at[slot],
                out_ref.at[pl.ds(origin * m_per, m_per), :],
                store_sems.at[slot],
            )
            st.start()
            return st

        st0 = compute_store(a_ref, my, 0)

        r1.wait_recv()
        r2.start()
        l1.wait_recv()
        l2.start()

        st1 = compute_store(cl_ref, left, 1)
        st0.wait()
        st2 = compute_store(cr_ref, right, 0)

        far = lax.rem(my + 2, N_DEV)
        r2.wait_recv()
        l2.wait_recv()
        st1.wait()
        st3 = compute_store(cf_ref, far, 1)

        st2.wait()
        st3.wait()
        r1.wait_send()
        l1.wait_send()
        r2.wait_send()
        l2.wait_send()

    return pl.pallas_call(
        body,
        out_shape=jax.ShapeDtypeStruct((N_DEV * m_per, n), jnp.bfloat16),
        in_specs=[
            pl.BlockSpec(memory_space=pltpu.MemorySpace.VMEM),
            pl.BlockSpec(memory_space=pltpu.MemorySpace.VMEM),
        ],
        out_specs=pl.BlockSpec(memory_space=pl.ANY),
        scratch_shapes=[
            pltpu.VMEM((m_per, k), jnp.bfloat16),
            pltpu.VMEM((m_per, k), jnp.bfloat16),
            pltpu.VMEM((m_per, k), jnp.bfloat16),
            pltpu.VMEM((m_per, k), jnp.bfloat16),
            pltpu.VMEM((2, m_per, n), jnp.bfloat16),
            pltpu.SemaphoreType.DMA((4,)),
            pltpu.SemaphoreType.DMA((4,)),
            pltpu.SemaphoreType.DMA((2,)),
        ],
        compiler_params=pltpu.CompilerParams(
            collective_id=0, vmem_limit_bytes=110 * 1024 * 1024
        ),
    )(A, B)
